# baseline (speedup 1.0000x reference)
import functools

import jax
import jax.numpy as jnp
from jax import lax
from jax.experimental import pallas as pl
from jax.experimental.pallas import tpu as pltpu
from jax.experimental.pallas import tpu_sc as plsc

NC, NS, L = 2, 16, 16
NW = NC * NS
D = 128
CHUNK = 64


def _emb_body(ids_hbm, tt_hbm, word_hbm, ttab_hbm, out_hbm,
              idx_v, tt_v, rows_v, ttab_v, gsem, wsem):
    wid = lax.axis_index("s") * NC + lax.axis_index("c")
    n_rows = idx_v.shape[0] // CHUNK
    tpw = n_rows * CHUNK
    s = ids_hbm.shape[1]
    w_per_b = s // tpw
    b = wid // w_per_b
    c0 = (wid % w_per_b) * tpw

    pltpu.sync_copy(ids_hbm.at[b, pl.ds(c0, tpw)], idx_v)
    for k in range(n_rows):
        pltpu.async_copy(word_hbm.at[idx_v.at[pl.ds(k * CHUNK, CHUNK)]],
                         rows_v.at[pl.ds(k * CHUNK, CHUNK)], gsem.at[k])

    pltpu.sync_copy(tt_hbm.at[b, pl.ds(c0, tpw)], tt_v)
    pltpu.sync_copy(ttab_hbm, ttab_v)

    tt0 = [ttab_v[0, pl.ds(j * L, L)] for j in range(D // L)]
    delta = [ttab_v[1, pl.ds(j * L, L)] - tt0[j] for j in range(D // L)]

    def chunk_body(k, _):
        off = k * CHUNK
        pltpu.make_async_copy(word_hbm.at[idx_v.at[pl.ds(off, CHUNK)]],
                              rows_v.at[pl.ds(off, CHUNK)],
                              gsem.at[k]).wait()

        def grp(g, _):
            i0 = off + g * L
            ttg = tt_v[pl.ds(i0, L)].astype(jnp.float32)
            for kk in range(L):
                i = i0 + kk
                sf = ttg[kk]
                for j in range(D // L):
                    sl = pl.ds(j * L, L)
                    rows_v[i, sl] = rows_v[i, sl] + tt0[j] + sf * delta[j]
            return 0

        lax.fori_loop(0, CHUNK // L, grp, 0)
        pltpu.async_copy(rows_v.at[pl.ds(off, CHUNK)],
                         out_hbm.at[b, pl.ds(c0 + off, CHUNK)],
                         wsem)
        return 0

    lax.fori_loop(0, n_rows, chunk_body, 0)

    for k in range(n_rows):
        pltpu.make_async_copy(rows_v.at[pl.ds(k * CHUNK, CHUNK)],
                              out_hbm.at[b, pl.ds(c0 + k * CHUNK, CHUNK)],
                              wsem).wait()


@jax.jit
def _emb(ids, tt, word, ttab):
    bsz, s = ids.shape
    n = bsz * s
    n_rows_w = n // NW // CHUNK
    mesh = plsc.VectorSubcoreMesh(core_axis_name="c", subcore_axis_name="s",
                                  num_cores=NC, num_subcores=NS)
    f = pl.kernel(
        _emb_body,
        out_type=jax.ShapeDtypeStruct((bsz, s, D), jnp.float32),
        mesh=mesh,
        scratch_types=[
            pltpu.VMEM((n_rows_w * CHUNK,), jnp.int32),
            pltpu.VMEM((n_rows_w * CHUNK,), jnp.int32),
            pltpu.VMEM((n_rows_w * CHUNK, D), jnp.float32),
            pltpu.VMEM((2, D), jnp.float32),
            pltpu.SemaphoreType.DMA((n_rows_w,)),
            pltpu.SemaphoreType.DMA,
        ],
    )
    return f(ids, tt, word, ttab)


def kernel(input_ids, token_type_ids, word_embeddings, token_type_embeddings):
    return _emb(input_ids.astype(jnp.int32), token_type_ids.astype(jnp.int32),
                word_embeddings, token_type_embeddings)


# speedup vs baseline: 1.0439x; 1.0439x over previous
import functools

import jax
import jax.numpy as jnp
from jax import lax
from jax.experimental import pallas as pl
from jax.experimental.pallas import tpu as pltpu
from jax.experimental.pallas import tpu_sc as plsc

NC, NS, L = 2, 16, 16
NW = NC * NS
D = 128
CHUNK = 128


def _emb_body(ids_hbm, tt_hbm, word_hbm, ttab_hbm, out_hbm,
              idx_v, tt_v, rows_v, ttab_v, gsem, wsem):
    wid = lax.axis_index("s") * NC + lax.axis_index("c")
    n_rows = idx_v.shape[0] // CHUNK
    tpw = n_rows * CHUNK
    s = ids_hbm.shape[1]
    w_per_b = s // tpw
    b = wid // w_per_b
    c0 = (wid % w_per_b) * tpw

    pltpu.sync_copy(ids_hbm.at[b, pl.ds(c0, tpw)], idx_v)
    for k in range(n_rows):
        pltpu.async_copy(word_hbm.at[idx_v.at[pl.ds(k * CHUNK, CHUNK)]],
                         rows_v.at[pl.ds(k * CHUNK, CHUNK)], gsem.at[k])

    pltpu.sync_copy(tt_hbm.at[b, pl.ds(c0, tpw)], tt_v)
    pltpu.sync_copy(ttab_hbm, ttab_v)

    tt0 = [ttab_v[0, pl.ds(j * L, L)] for j in range(D // L)]
    delta = [ttab_v[1, pl.ds(j * L, L)] - tt0[j] for j in range(D // L)]

    def chunk_body(k, _):
        off = k * CHUNK
        pltpu.make_async_copy(word_hbm.at[idx_v.at[pl.ds(off, CHUNK)]],
                              rows_v.at[pl.ds(off, CHUNK)],
                              gsem.at[k]).wait()

        def grp(g, _):
            i0 = off + g * L
            ttg = tt_v[pl.ds(i0, L)].astype(jnp.float32)
            for kk in range(L):
                i = i0 + kk
                sf = ttg[kk]
                for j in range(D // L):
                    sl = pl.ds(j * L, L)
                    rows_v[i, sl] = rows_v[i, sl] + tt0[j] + sf * delta[j]
            return 0

        lax.fori_loop(0, CHUNK // L, grp, 0)
        pltpu.async_copy(rows_v.at[pl.ds(off, CHUNK)],
                         out_hbm.at[b, pl.ds(c0 + off, CHUNK)],
                         wsem)
        return 0

    lax.fori_loop(0, n_rows, chunk_body, 0)

    for k in range(n_rows):
        pltpu.make_async_copy(rows_v.at[pl.ds(k * CHUNK, CHUNK)],
                              out_hbm.at[b, pl.ds(c0 + k * CHUNK, CHUNK)],
                              wsem).wait()


@jax.jit
def _emb(ids, tt, word, ttab):
    bsz, s = ids.shape
    n = bsz * s
    n_rows_w = n // NW // CHUNK
    mesh = plsc.VectorSubcoreMesh(core_axis_name="c", subcore_axis_name="s",
                                  num_cores=NC, num_subcores=NS)
    f = pl.kernel(
        _emb_body,
        out_type=jax.ShapeDtypeStruct((bsz, s, D), jnp.float32),
        mesh=mesh,
        scratch_types=[
            pltpu.VMEM((n_rows_w * CHUNK,), jnp.int32),
            pltpu.VMEM((n_rows_w * CHUNK,), jnp.int32),
            pltpu.VMEM((n_rows_w * CHUNK, D), jnp.float32),
            pltpu.VMEM((2, D), jnp.float32),
            pltpu.SemaphoreType.DMA((n_rows_w,)),
            pltpu.SemaphoreType.DMA,
        ],
    )
    return f(ids, tt, word, ttab)


def kernel(input_ids, token_type_ids, word_embeddings, token_type_embeddings):
    return _emb(input_ids.astype(jnp.int32), token_type_ids.astype(jnp.int32),
                word_embeddings, token_type_embeddings)
